# baseline (speedup 1.0000x reference)
import functools

import jax
import jax.numpy as jnp
from jax import lax
from jax.experimental import pallas as pl
from jax.experimental.pallas import tpu as pltpu
from jax.experimental.pallas import tpu_sc as plsc

M_ROWS = 1_000_000
HALF = M_ROWS // 2
D = 64
B = 16384
NTILES = 16
EPT = B // NTILES
CH = 128
NCH = EPT // CH
LANES = 16

_mesh = plsc.VectorSubcoreMesh(core_axis_name="c", subcore_axis_name="s")


def _sc_body(mem_hbm, idx_hbm, val_hbm, out_hbm,
             rep_t, acc_s, idx_v, r_v, o_v, pos_v, t_v, val_v, g_v, m_v, sem):
    tid = lax.axis_index("s")

    if True:
        ebase = tid * EPT

        for c in range(NCH):
            pltpu.sync_copy(idx_hbm.at[pl.ds(ebase + c * CH, CH)],
                            idx_v.at[c])

        for c in range(NCH):
            def _fill_pos(j, _):
                pos_v[c, pl.ds(j * LANES, LANES)] = (
                    lax.iota(jnp.int32, LANES) + (ebase + c * CH + j * LANES))
                return 0
            lax.fori_loop(0, CH // LANES, _fill_pos, 0)

        def _zero_row(i, _):
            for q in range(D // LANES):
                val_v[i, pl.ds(q * LANES, LANES)] = jnp.zeros(
                    (LANES,), jnp.float32)
            return 0
        lax.fori_loop(0, CH, _zero_row, 0)
        for c in range(NCH):
            pltpu.sync_copy(val_v, acc_s.at[pl.ds(ebase + c * CH, CH)])

        for h in range(2):
            base = h * HALF

            for c in range(NCH):
                def _mk_off(j, _):
                    sl = pl.ds(j * LANES, LANES)
                    iv = idx_v[c, sl]
                    ok = (iv >= base) & (iv < base + HALF)
                    o_v[c, sl] = jnp.where(ok, iv - base, HALF + tid)
                    return 0
                lax.fori_loop(0, CH // LANES, _mk_off, 0)
                pltpu.sync_copy(pos_v.at[c], rep_t.at[o_v.at[c]])
            plsc.subcore_barrier()
            for c in range(NCH):
                pltpu.sync_copy(rep_t.at[o_v.at[c]], t_v)

                def _merge(j, _):
                    sl = pl.ds(j * LANES, LANES)
                    iv = idx_v[c, sl]
                    ok = (iv >= base) & (iv < base + HALF)
                    r_v[c, sl] = jnp.where(ok, t_v[sl], r_v[c, sl])
                    return 0
                lax.fori_loop(0, CH // LANES, _merge, 0)
            plsc.subcore_barrier()

        for c in range(NCH):
            pltpu.sync_copy(val_hbm.at[pl.ds(ebase + c * CH, CH)], val_v)
            pltpu.sync_copy(val_v, acc_s.at[r_v.at[c]], add=True)
        plsc.subcore_barrier()

        for c in range(NCH):
            pltpu.async_copy(mem_hbm.at[idx_v.at[c]], m_v, sem).wait()
            pltpu.sync_copy(acc_s.at[r_v.at[c]], g_v)

            def _add_row(i, _):
                for q in range(D // LANES):
                    sl = pl.ds(q * LANES, LANES)
                    g_v[i, sl] = g_v[i, sl] + m_v[i, sl]
                return 0
            lax.fori_loop(0, CH, _add_row, 0)
            pltpu.sync_copy(g_v, out_hbm.at[pl.ds(ebase + c * CH, CH)])


_sc_call = functools.partial(
    pl.kernel,
    out_type=jax.ShapeDtypeStruct((B, D), jnp.float32),
    mesh=_mesh,
    scratch_types=[
        pltpu.VMEM_SHARED((HALF + NTILES,), jnp.int32),
        pltpu.VMEM_SHARED((B, D), jnp.float32),
        pltpu.VMEM((NCH, CH), jnp.int32),
        pltpu.VMEM((NCH, CH), jnp.int32),
        pltpu.VMEM((NCH, CH), jnp.int32),
        pltpu.VMEM((NCH, CH), jnp.int32),
        pltpu.VMEM((CH,), jnp.int32),
        pltpu.VMEM((CH, D), jnp.float32),
        pltpu.VMEM((CH, D), jnp.float32),
        pltpu.VMEM((CH, D), jnp.float32),
        pltpu.SemaphoreType.DMA,
    ],
    compiler_params=pltpu.CompilerParams(use_tc_tiling_on_sc=False),
)(_sc_body)


@jax.jit
def kernel(mem, idx, val):
    return _sc_call(mem, idx, val)


# speedup vs baseline: 1.0050x; 1.0050x over previous
import functools

import jax
import jax.numpy as jnp
from jax import lax
from jax.experimental import pallas as pl
from jax.experimental.pallas import tpu as pltpu
from jax.experimental.pallas import tpu_sc as plsc

M_ROWS = 1_000_000
HALF = M_ROWS // 2
D = 64
B = 16384
NTILES = 16
EPT = B // NTILES
CH = 128
NCH = EPT // CH
LANES = 16

_mesh = plsc.VectorSubcoreMesh(core_axis_name="c", subcore_axis_name="s")


def _sc_body(mem_hbm, idx_hbm, val_hbm, out_hbm,
             rep_t, acc_s, idx_v, r_v, o_v, pos_v, t_v, val_v, g_v, m_v, sem):
    core = lax.axis_index("c")
    tid = lax.axis_index("s")

    @pl.when(core == 0)
    def _():
        ebase = tid * EPT

        for c in range(NCH):
            pltpu.sync_copy(idx_hbm.at[pl.ds(ebase + c * CH, CH)],
                            idx_v.at[c])

        for c in range(NCH):
            def _fill_pos(j, _):
                pos_v[c, pl.ds(j * LANES, LANES)] = (
                    lax.iota(jnp.int32, LANES) + (ebase + c * CH + j * LANES))
                return 0
            lax.fori_loop(0, CH // LANES, _fill_pos, 0)

        def _zero_row(i, _):
            for q in range(D // LANES):
                val_v[i, pl.ds(q * LANES, LANES)] = jnp.zeros(
                    (LANES,), jnp.float32)
            return 0
        lax.fori_loop(0, CH, _zero_row, 0)
        for c in range(NCH):
            pltpu.sync_copy(val_v, acc_s.at[pl.ds(ebase + c * CH, CH)])

        for h in range(2):
            base = h * HALF

            for c in range(NCH):
                def _mk_off(j, _):
                    sl = pl.ds(j * LANES, LANES)
                    iv = idx_v[c, sl]
                    ok = (iv >= base) & (iv < base + HALF)
                    o_v[c, sl] = jnp.where(ok, iv - base, HALF + tid)
                    return 0
                lax.fori_loop(0, CH // LANES, _mk_off, 0)
                pltpu.sync_copy(pos_v.at[c], rep_t.at[o_v.at[c]])
            plsc.subcore_barrier()
            for c in range(NCH):
                pltpu.sync_copy(rep_t.at[o_v.at[c]], t_v)

                def _merge(j, _):
                    sl = pl.ds(j * LANES, LANES)
                    iv = idx_v[c, sl]
                    ok = (iv >= base) & (iv < base + HALF)
                    r_v[c, sl] = jnp.where(ok, t_v[sl], r_v[c, sl])
                    return 0
                lax.fori_loop(0, CH // LANES, _merge, 0)
            plsc.subcore_barrier()

        for c in range(NCH):
            pltpu.sync_copy(val_hbm.at[pl.ds(ebase + c * CH, CH)], val_v)
            pltpu.sync_copy(val_v, acc_s.at[r_v.at[c]], add=True)
        plsc.subcore_barrier()

        for c in range(NCH):
            pltpu.async_copy(mem_hbm.at[idx_v.at[c]], m_v, sem).wait()
            pltpu.sync_copy(acc_s.at[r_v.at[c]], g_v)

            def _add_row(i, _):
                for q in range(D // LANES):
                    sl = pl.ds(q * LANES, LANES)
                    g_v[i, sl] = g_v[i, sl] + m_v[i, sl]
                return 0
            lax.fori_loop(0, CH, _add_row, 0)
            pltpu.sync_copy(g_v, out_hbm.at[pl.ds(ebase + c * CH, CH)])


_sc_call = functools.partial(
    pl.kernel,
    out_type=jax.ShapeDtypeStruct((B, D), jnp.float32),
    mesh=_mesh,
    scratch_types=[
        pltpu.VMEM_SHARED((HALF + NTILES,), jnp.int32),
        pltpu.VMEM_SHARED((B, D), jnp.float32),
        pltpu.VMEM((NCH, CH), jnp.int32),
        pltpu.VMEM((NCH, CH), jnp.int32),
        pltpu.VMEM((NCH, CH), jnp.int32),
        pltpu.VMEM((NCH, CH), jnp.int32),
        pltpu.VMEM((CH,), jnp.int32),
        pltpu.VMEM((CH, D), jnp.float32),
        pltpu.VMEM((CH, D), jnp.float32),
        pltpu.VMEM((CH, D), jnp.float32),
        pltpu.SemaphoreType.DMA,
    ],
    compiler_params=pltpu.CompilerParams(use_tc_tiling_on_sc=False),
)(_sc_body)


@jax.jit
def kernel(mem, idx, val):
    return _sc_call(mem, idx, val)
